# baseline (speedup 1.0000x reference)
import functools

import jax
import jax.numpy as jnp
from jax import lax
from jax.experimental import pallas as pl
from jax.experimental.pallas import tpu as pltpu
from jax.experimental.pallas import tpu_sc as plsc

DIM = 64
LANES = 16


def _tree_sum_kernel(B, bpw, NC):
    mesh = plsc.VectorSubcoreMesh(core_axis_name="c", subcore_axis_name="s")

    @functools.partial(
        pl.kernel,
        mesh=mesh,
        compiler_params=pltpu.CompilerParams(use_tc_tiling_on_sc=False),
        out_type=jax.ShapeDtypeStruct((B, DIM), jnp.float32),
        scratch_types=[
            pltpu.VMEM((bpw,), jnp.int32),
            pltpu.VMEM((bpw,), jnp.int32),
            pltpu.VMEM((bpw,), jnp.int32),
            pltpu.VMEM((bpw,), jnp.int32),
            pltpu.VMEM((bpw, DIM), jnp.float32),
            pltpu.VMEM((bpw, DIM), jnp.float32),
            pltpu.VMEM((bpw, DIM), jnp.float32),
            pltpu.VMEM((bpw, DIM), jnp.float32),
            pltpu.SemaphoreType.DMA,
        ],
    )
    def k(idx_hbm, t0, t1, t2, t3, p0, p1, p2, out_hbm,
          i0, i1, i2, i3, buf0, buf1, buf2, buf3, sem):
        wid = lax.axis_index("s") * NC + lax.axis_index("c")
        base = wid * bpw

        with jax.named_scope("ph_idx"):
            pltpu.sync_copy(idx_hbm.at[pl.ds(base, bpw)], i0)
        with jax.named_scope("ph_chain"):
            c0 = pltpu.async_copy(t0.at[i0], buf0, sem)
            pltpu.sync_copy(p0.at[i0], i1)
            c1 = pltpu.async_copy(t1.at[i1], buf1, sem)
            pltpu.sync_copy(p1.at[i1], i2)
            c2 = pltpu.async_copy(t2.at[i2], buf2, sem)
            pltpu.sync_copy(p2.at[i2], i3)
            c3 = pltpu.async_copy(t3.at[i3], buf3, sem)
        with jax.named_scope("ph_drain"):
            c0.wait()
            c1.wait()
            c2.wait()
            c3.wait()

        def row(r, _):
            for j in range(DIM // LANES):
                d = pl.ds(j * LANES, LANES)
                buf0[r, d] = (buf0[r, d] + buf1[r, d]) + (buf2[r, d] + buf3[r, d])
            return 0

        with jax.named_scope("ph_acc"):
            lax.fori_loop(0, bpw, row, 0)
        with jax.named_scope("ph_out"):
            pltpu.sync_copy(buf0, out_hbm.at[pl.ds(base, bpw)])

    return k


def kernel(idx, T0, T1, T2, T3, P0, P1, P2):
    B = idx.shape[0]
    info = plsc.get_sparse_core_info()
    NC = info.num_cores
    NW = NC * info.num_subcores
    bpw = B // NW
    k = _tree_sum_kernel(B, bpw, NC)
    return k(idx.astype(jnp.int32), T0, T1, T2, T3, P0, P1, P2)


# speedup vs baseline: 1.0069x; 1.0069x over previous
import functools

import jax
import jax.numpy as jnp
from jax import lax
from jax.experimental import pallas as pl
from jax.experimental.pallas import tpu as pltpu
from jax.experimental.pallas import tpu_sc as plsc

DIM = 64
PAD = 128
LANES = 16


def _tree_sum_kernel(B, bpw, NC):
    mesh = plsc.VectorSubcoreMesh(core_axis_name="c", subcore_axis_name="s")

    @functools.partial(
        pl.kernel,
        mesh=mesh,
        compiler_params=pltpu.CompilerParams(use_tc_tiling_on_sc=False),
        out_type=jax.ShapeDtypeStruct((B, DIM), jnp.float32),
        scratch_types=[
            pltpu.VMEM((bpw,), jnp.int32),
            pltpu.VMEM((bpw,), jnp.int32),
            pltpu.VMEM((bpw,), jnp.int32),
            pltpu.VMEM((bpw,), jnp.int32),
            pltpu.VMEM((bpw, PAD), jnp.float32),
            pltpu.VMEM((bpw, PAD), jnp.float32),
            pltpu.VMEM((bpw, PAD), jnp.float32),
            pltpu.VMEM((bpw, PAD), jnp.float32),
            pltpu.VMEM((bpw, DIM), jnp.float32),
            pltpu.SemaphoreType.DMA,
        ],
    )
    def k(idx_hbm, t0, t1, t2, t3, p0, p1, p2, out_hbm,
          i0, i1, i2, i3, buf0, buf1, buf2, buf3, obuf, sem):
        wid = lax.axis_index("s") * NC + lax.axis_index("c")
        base = wid * bpw

        pltpu.sync_copy(idx_hbm.at[pl.ds(base, bpw)], i0)
        c0 = pltpu.async_copy(t0.at[i0], buf0, sem)
        pltpu.sync_copy(p0.at[i0], i1)
        c1 = pltpu.async_copy(t1.at[i1], buf1, sem)
        pltpu.sync_copy(p1.at[i1], i2)
        c2 = pltpu.async_copy(t2.at[i2], buf2, sem)
        pltpu.sync_copy(p2.at[i2], i3)
        c3 = pltpu.async_copy(t3.at[i3], buf3, sem)
        c0.wait()
        c1.wait()
        c2.wait()
        c3.wait()

        def row(r, _):
            for j in range(DIM // LANES):
                d = pl.ds(j * LANES, LANES)
                obuf[r, d] = (buf0[r, d] + buf1[r, d]) + (buf2[r, d] + buf3[r, d])
            return 0

        lax.fori_loop(0, bpw, row, 0)
        pltpu.sync_copy(obuf, out_hbm.at[pl.ds(base, bpw)])

    return k


def kernel(idx, T0, T1, T2, T3, P0, P1, P2):
    B = idx.shape[0]
    info = plsc.get_sparse_core_info()
    NC = info.num_cores
    NW = NC * info.num_subcores
    bpw = B // NW
    pad = ((0, 0), (0, PAD - DIM))
    k = _tree_sum_kernel(B, bpw, NC)
    return k(idx.astype(jnp.int32),
             jnp.pad(T0, pad), jnp.pad(T1, pad), jnp.pad(T2, pad),
             jnp.pad(T3, pad), P0, P1, P2)


# speedup vs baseline: 1.2248x; 1.2165x over previous
import functools

import jax
import jax.numpy as jnp
from jax import lax
from jax.experimental import pallas as pl
from jax.experimental.pallas import tpu as pltpu
from jax.experimental.pallas import tpu_sc as plsc

DIM = 64
PAD = 128
LANES = 16

N1 = 3125
N2 = 97
N3 = 3
VSLAB = 208
NP0 = 100000
P0SLICE = 6256
P0LAST = 6160


def _tree_sum_kernel(B, bpw, NC, NS):
    mesh = plsc.VectorSubcoreMesh(core_axis_name="c", subcore_axis_name="s")

    @functools.partial(
        pl.kernel,
        mesh=mesh,
        compiler_params=pltpu.CompilerParams(
            use_tc_tiling_on_sc=False, needs_layout_passes=False),
        out_type=jax.ShapeDtypeStruct((B, DIM), jnp.float32),
        scratch_types=[
            pltpu.VMEM((bpw,), jnp.int32),
            pltpu.VMEM((bpw,), jnp.int32),
            pltpu.VMEM((bpw, PAD), jnp.float32),
            pltpu.VMEM((bpw, DIM), jnp.float32),
            pltpu.VMEM((VSLAB, DIM), jnp.float32),
            pltpu.VMEM((VSLAB, DIM), jnp.float32),
            pltpu.VMEM((N1,), jnp.int32),
            pltpu.VMEM((N2,), jnp.int32),
            pltpu.VMEM((N2, DIM), jnp.float32),
            pltpu.VMEM((N3, DIM), jnp.float32),
            pltpu.VMEM_SHARED((NS * P0SLICE,), jnp.int32),
            pltpu.VMEM_SHARED((NS * VSLAB, DIM), jnp.float32),
            pltpu.SemaphoreType.DMA,
            pltpu.SemaphoreType.DMA,
        ],
    )
    def k(idx_hbm, t0, t1, t2, t3, p0, p1, p2, out_hbm,
          i0, i1, buf0, bufu, t1slab, u1slab, p1buf, p2buf, t2buf, t3buf,
          p0s, u1s, sem0, sem1):
        c = lax.axis_index("c")
        s = lax.axis_index("s")
        wid = s * NC + c
        base = wid * bpw

        with jax.named_scope("ph_idx"):
            pltpu.sync_copy(idx_hbm.at[pl.ds(base, bpw)], i0)
        with jax.named_scope("ph_t0"):
            c_t0 = pltpu.async_copy(t0.at[i0], buf0, sem0)

        with jax.named_scope("ph_stage"):
            sbase = s * P0SLICE

            @pl.when(s < NS - 1)
            def _():
                pltpu.sync_copy(p0.at[pl.ds(sbase, P0SLICE)],
                                p0s.at[pl.ds(sbase, P0SLICE)])

            @pl.when(s == NS - 1)
            def _():
                pltpu.sync_copy(p0.at[pl.ds(sbase, P0LAST)],
                                p0s.at[pl.ds(sbase, P0LAST)])

            pltpu.sync_copy(p1, p1buf)
            pltpu.sync_copy(p2, p2buf)
            pltpu.sync_copy(t2, t2buf)
            pltpu.sync_copy(t3, t3buf)
            vbase = s * VSLAB

            @pl.when(s < NS - 1)
            def _():
                pltpu.sync_copy(t1.at[pl.ds(vbase, VSLAB)], t1slab)

            @pl.when(s == NS - 1)
            def _():
                pltpu.sync_copy(t1.at[pl.ds((NS - 1) * VSLAB, N1 - (NS - 1) * VSLAB)],
                                t1slab.at[pl.ds(0, N1 - (NS - 1) * VSLAB)])

        with jax.named_scope("ph_build"):
            lanes = lax.iota(jnp.int32, LANES)

            def group(g, _):
                lrow = g * LANES + lanes
                v = jnp.minimum(vbase + lrow, N1 - 1)
                j2 = plsc.load_gather(p1buf, [v])
                j2 = jnp.minimum(jnp.maximum(j2, 0), N2 - 1)
                j3 = plsc.load_gather(p2buf, [j2])
                j3 = jnp.minimum(jnp.maximum(j3, 0), N3 - 1)
                for col in range(DIM):
                    cv = jnp.full((LANES,), col, jnp.int32)
                    acc = (plsc.load_gather(t1slab, [lrow, cv])
                           + plsc.load_gather(t2buf, [j2, cv])
                           + plsc.load_gather(t3buf, [j3, cv]))
                    plsc.store_scatter(u1slab, [lrow, cv], acc)
                return 0

            lax.fori_loop(0, VSLAB // LANES, group, 0)
            pltpu.sync_copy(u1slab, u1s.at[pl.ds(vbase, VSLAB)])

        with jax.named_scope("ph_barrier"):
            plsc.subcore_barrier()

        with jax.named_scope("ph_chain"):
            pltpu.async_copy(p0s.at[i0], i1, sem1).wait()
            c_u = pltpu.async_copy(u1s.at[i1], bufu, sem1)
            c_u.wait()
            c_t0.wait()

        with jax.named_scope("ph_acc"):
            def row(r, _):
                for j in range(DIM // LANES):
                    d = pl.ds(j * LANES, LANES)
                    bufu[r, d] = bufu[r, d] + buf0[r, d]
                return 0

            lax.fori_loop(0, bpw, row, 0)

        with jax.named_scope("ph_out"):
            pltpu.sync_copy(bufu, out_hbm.at[pl.ds(base, bpw)])

    return k


def kernel(idx, T0, T1, T2, T3, P0, P1, P2):
    B = idx.shape[0]
    info = plsc.get_sparse_core_info()
    NC = info.num_cores
    NS = info.num_subcores
    bpw = B // (NC * NS)
    k = _tree_sum_kernel(B, bpw, NC, NS)
    T0p = jnp.pad(T0, ((0, 0), (0, PAD - DIM)))
    return k(idx.astype(jnp.int32), T0p, T1, T2, T3, P0, P1, P2)


# speedup vs baseline: 1.6319x; 1.3323x over previous
import functools

import jax
import jax.numpy as jnp
from jax import lax
from jax.experimental import pallas as pl
from jax.experimental.pallas import tpu as pltpu
from jax.experimental.pallas import tpu_sc as plsc

DIM = 64
PAD = 128
LANES = 16

N1 = 3125
N2 = 97
N3 = 3
VSLAB = 208
VPAD = 3328
N1TAIL = N1 - 15 * VSLAB
NP0 = 100000
P0SLICE = 6256
P0LAST = 6160


def _tree_sum_kernel(B, bpw, NC, NS):
    mesh = plsc.VectorSubcoreMesh(core_axis_name="c", subcore_axis_name="s")

    @functools.partial(
        pl.kernel,
        mesh=mesh,
        compiler_params=pltpu.CompilerParams(
            use_tc_tiling_on_sc=False, needs_layout_passes=False),
        out_type=jax.ShapeDtypeStruct((B, DIM), jnp.float32),
        scratch_types=[
            pltpu.VMEM((bpw,), jnp.int32),
            pltpu.VMEM((bpw,), jnp.int32),
            pltpu.VMEM((bpw, PAD), jnp.float32),
            pltpu.VMEM((bpw, DIM), jnp.float32),
            pltpu.VMEM((VSLAB, DIM), jnp.float32),
            pltpu.VMEM((VSLAB, DIM), jnp.float32),
            pltpu.VMEM((VSLAB, DIM), jnp.float32),
            pltpu.VMEM((VPAD,), jnp.int32),
            pltpu.VMEM((VSLAB,), jnp.int32),
            pltpu.VMEM((VSLAB,), jnp.int32),
            pltpu.VMEM_SHARED((NS * P0SLICE,), jnp.int32),
            pltpu.VMEM_SHARED((VPAD, DIM), jnp.float32),
            pltpu.VMEM_SHARED((N2,), jnp.int32),
            pltpu.VMEM_SHARED((N2, DIM), jnp.float32),
            pltpu.VMEM_SHARED((N3, DIM), jnp.float32),
            pltpu.SemaphoreType.DMA,
            pltpu.SemaphoreType.DMA,
        ],
    )
    def k(idx_hbm, t0, t1, t2, t3, p0, p1, p2, out_hbm,
          i0, i1, buf0, bufu, t1slab, t2rows, t3rows, p1full, j3buf, ident,
          p0s, u1s, p2s, t2s, t3s, sem0, sem1):
        c = lax.axis_index("c")
        s = lax.axis_index("s")
        wid = s * NC + c
        base = wid * bpw
        vbase = s * VSLAB
        lanes = lax.iota(jnp.int32, LANES)

        with jax.named_scope("ph_idx"):
            pltpu.sync_copy(idx_hbm.at[pl.ds(base, bpw)], i0)
        with jax.named_scope("ph_t0"):
            c_t0 = pltpu.async_copy(t0.at[i0], buf0, sem0)

        with jax.named_scope("ph_stage"):
            sbase = s * P0SLICE

            @pl.when(s < NS - 1)
            def _():
                pltpu.sync_copy(p0.at[pl.ds(sbase, P0SLICE)],
                                p0s.at[pl.ds(sbase, P0SLICE)])
                pltpu.sync_copy(t1.at[pl.ds(vbase, VSLAB)], t1slab)

            @pl.when(s == NS - 1)
            def _():
                pltpu.sync_copy(p0.at[pl.ds(sbase, P0LAST)],
                                p0s.at[pl.ds(sbase, P0LAST)])
                pltpu.sync_copy(t1.at[pl.ds(15 * VSLAB, N1TAIL)],
                                t1slab.at[pl.ds(0, N1TAIL)])

            pltpu.sync_copy(p1, p1full.at[pl.ds(0, N1)])

            @pl.when(s == 0)
            def _():
                pltpu.sync_copy(p2, p2s)
                pltpu.sync_copy(t2, t2s)
                pltpu.sync_copy(t3, t3s)

        with jax.named_scope("ph_prep"):
            for g in range(VSLAB // LANES):
                d = pl.ds(g * LANES, LANES)
                gl = vbase + g * LANES + lanes
                v = p1full[pl.ds(vbase + g * LANES, LANES)]
                p1full[pl.ds(vbase + g * LANES, LANES)] = (
                    jnp.minimum(jnp.maximum(v, 0), N2 - 1))
                ident[d] = gl

        with jax.named_scope("ph_barrier1"):
            plsc.subcore_barrier()

        with jax.named_scope("ph_build"):
            myp1 = p1full.at[pl.ds(vbase, VSLAB)]
            pltpu.async_copy(p2s.at[myp1], j3buf, sem1).wait()
            for g in range(VSLAB // LANES):
                d = pl.ds(g * LANES, LANES)
                j3 = j3buf[d]
                j3buf[d] = jnp.minimum(jnp.maximum(j3, 0), N3 - 1)
            c_a = pltpu.async_copy(t2s.at[myp1], t2rows, sem1)
            c_b = pltpu.async_copy(t3s.at[j3buf], t3rows, sem1)
            c_a.wait()
            c_b.wait()
            pltpu.sync_copy(t1slab, u1s.at[pl.ds(vbase, VSLAB)])
            pltpu.sync_copy(t2rows, u1s.at[ident], add=True)
            pltpu.sync_copy(t3rows, u1s.at[ident], add=True)

        with jax.named_scope("ph_barrier2"):
            plsc.subcore_barrier()

        with jax.named_scope("ph_chain"):
            pltpu.async_copy(p0s.at[i0], i1, sem1).wait()
            pltpu.async_copy(u1s.at[i1], bufu, sem1).wait()
            c_t0.wait()

        with jax.named_scope("ph_acc"):
            def row(r, _):
                for j in range(DIM // LANES):
                    d = pl.ds(j * LANES, LANES)
                    bufu[r, d] = bufu[r, d] + buf0[r, d]
                return 0

            lax.fori_loop(0, bpw, row, 0, unroll=2)

        with jax.named_scope("ph_out"):
            pltpu.sync_copy(bufu, out_hbm.at[pl.ds(base, bpw)])

    return k


def kernel(idx, T0, T1, T2, T3, P0, P1, P2):
    B = idx.shape[0]
    info = plsc.get_sparse_core_info()
    NC = info.num_cores
    NS = info.num_subcores
    bpw = B // (NC * NS)
    k = _tree_sum_kernel(B, bpw, NC, NS)
    T0p = jnp.pad(T0, ((0, 0), (0, PAD - DIM)))
    return k(idx.astype(jnp.int32), T0p, T1, T2, T3, P0, P1, P2)


# speedup vs baseline: 1.7619x; 1.0797x over previous
import functools

import jax
import jax.numpy as jnp
from jax import lax
from jax.experimental import pallas as pl
from jax.experimental.pallas import tpu as pltpu
from jax.experimental.pallas import tpu_sc as plsc

DIM = 64
PAD = 128
LANES = 16

N1 = 3125
N2 = 97
N3 = 3
VSLAB = 208
VPAD = 3328
N1TAIL = N1 - 15 * VSLAB
P0SLICE = 6256
P0LAST = 6160

_params = pltpu.CompilerParams(
    use_tc_tiling_on_sc=False, needs_layout_passes=False)


def _upper_kernel(B, bpw, NC, NS):
    mesh = plsc.VectorSubcoreMesh(core_axis_name="c", subcore_axis_name="s")

    @functools.partial(
        pl.kernel,
        mesh=mesh,
        compiler_params=_params,
        out_type=jax.ShapeDtypeStruct((B, DIM), jnp.float32),
        scratch_types=[
            pltpu.VMEM((bpw,), jnp.int32),
            pltpu.VMEM((bpw,), jnp.int32),
            pltpu.VMEM((bpw, DIM), jnp.float32),
            pltpu.VMEM((VSLAB, DIM), jnp.float32),
            pltpu.VMEM((VSLAB, DIM), jnp.float32),
            pltpu.VMEM((VSLAB, DIM), jnp.float32),
            pltpu.VMEM((VPAD,), jnp.int32),
            pltpu.VMEM((VSLAB,), jnp.int32),
            pltpu.VMEM((VSLAB,), jnp.int32),
            pltpu.VMEM_SHARED((NS * P0SLICE,), jnp.int32),
            pltpu.VMEM_SHARED((VPAD, DIM), jnp.float32),
            pltpu.VMEM_SHARED((N2,), jnp.int32),
            pltpu.VMEM_SHARED((N2, DIM), jnp.float32),
            pltpu.VMEM_SHARED((N3, DIM), jnp.float32),
            pltpu.SemaphoreType.DMA,
        ],
    )
    def ka(idx_hbm, t1, t2, t3, p0, p1, p2, out_hbm,
           i0, i1, bufu, t1slab, t2rows, t3rows, p1full, j3buf, ident,
           p0s, u1s, p2s, t2s, t3s, sem):
        c = lax.axis_index("c")
        s = lax.axis_index("s")
        base = (s * NC + c) * bpw
        vbase = s * VSLAB
        lanes = lax.iota(jnp.int32, LANES)

        with jax.named_scope("ph_idx"):
            pltpu.sync_copy(idx_hbm.at[pl.ds(base, bpw)], i0)

        with jax.named_scope("ph_stage"):
            sbase = s * P0SLICE

            @pl.when(s < NS - 1)
            def _():
                pltpu.sync_copy(p0.at[pl.ds(sbase, P0SLICE)],
                                p0s.at[pl.ds(sbase, P0SLICE)])
                pltpu.sync_copy(t1.at[pl.ds(vbase, VSLAB)], t1slab)

            @pl.when(s == NS - 1)
            def _():
                pltpu.sync_copy(p0.at[pl.ds(sbase, P0LAST)],
                                p0s.at[pl.ds(sbase, P0LAST)])
                pltpu.sync_copy(t1.at[pl.ds(15 * VSLAB, N1TAIL)],
                                t1slab.at[pl.ds(0, N1TAIL)])

            pltpu.sync_copy(p1, p1full.at[pl.ds(0, N1)])

            @pl.when(s == 0)
            def _():
                pltpu.sync_copy(p2, p2s)
                pltpu.sync_copy(t2, t2s)
                pltpu.sync_copy(t3, t3s)

        with jax.named_scope("ph_prep"):
            for g in range(VSLAB // LANES):
                d = pl.ds(g * LANES, LANES)
                v = p1full[pl.ds(vbase + g * LANES, LANES)]
                p1full[pl.ds(vbase + g * LANES, LANES)] = (
                    jnp.minimum(jnp.maximum(v, 0), N2 - 1))
                ident[d] = vbase + g * LANES + lanes

        with jax.named_scope("ph_barrier1"):
            plsc.subcore_barrier()

        with jax.named_scope("ph_build"):
            myp1 = p1full.at[pl.ds(vbase, VSLAB)]
            pltpu.async_copy(p2s.at[myp1], j3buf, sem).wait()
            for g in range(VSLAB // LANES):
                d = pl.ds(g * LANES, LANES)
                j3 = j3buf[d]
                j3buf[d] = jnp.minimum(jnp.maximum(j3, 0), N3 - 1)
            c_a = pltpu.async_copy(t2s.at[myp1], t2rows, sem)
            c_b = pltpu.async_copy(t3s.at[j3buf], t3rows, sem)
            c_a.wait()
            c_b.wait()
            pltpu.sync_copy(t1slab, u1s.at[pl.ds(vbase, VSLAB)])
            pltpu.sync_copy(t2rows, u1s.at[ident], add=True)
            pltpu.sync_copy(t3rows, u1s.at[ident], add=True)

        with jax.named_scope("ph_barrier2"):
            plsc.subcore_barrier()

        with jax.named_scope("ph_chain"):
            pltpu.async_copy(p0s.at[i0], i1, sem).wait()
            pltpu.async_copy(u1s.at[i1], bufu, sem).wait()

        with jax.named_scope("ph_outA"):
            pltpu.sync_copy(bufu, out_hbm.at[pl.ds(base, bpw)])

    return ka


def _leaf_kernel(B, bpw, NC):
    mesh = plsc.VectorSubcoreMesh(core_axis_name="c", subcore_axis_name="s")

    @functools.partial(
        pl.kernel,
        mesh=mesh,
        compiler_params=_params,
        out_type=jax.ShapeDtypeStruct((B, DIM), jnp.float32),
        scratch_types=[
            pltpu.VMEM((bpw,), jnp.int32),
            pltpu.VMEM((bpw, PAD), jnp.float32),
            pltpu.VMEM((bpw, DIM), jnp.float32),
            pltpu.SemaphoreType.DMA,
        ],
    )
    def kb(idx_hbm, t0, part_hbm, out_hbm, i0, buf0, bufu, sem):
        c = lax.axis_index("c")
        s = lax.axis_index("s")
        base = (s * NC + c) * bpw

        with jax.named_scope("ph_idxB"):
            pltpu.sync_copy(idx_hbm.at[pl.ds(base, bpw)], i0)
        with jax.named_scope("ph_t0"):
            c_t0 = pltpu.async_copy(t0.at[i0], buf0, sem)
            pltpu.sync_copy(part_hbm.at[pl.ds(base, bpw)], bufu)
            c_t0.wait()

        with jax.named_scope("ph_acc"):
            def row(r, _):
                for j in range(DIM // LANES):
                    d = pl.ds(j * LANES, LANES)
                    bufu[r, d] = bufu[r, d] + buf0[r, d]
                return 0

            lax.fori_loop(0, bpw, row, 0, unroll=2)

        with jax.named_scope("ph_outB"):
            pltpu.sync_copy(bufu, out_hbm.at[pl.ds(base, bpw)])

    return kb


def kernel(idx, T0, T1, T2, T3, P0, P1, P2):
    B = idx.shape[0]
    info = plsc.get_sparse_core_info()
    NC = info.num_cores
    NS = info.num_subcores
    bpw = B // (NC * NS)
    idx32 = idx.astype(jnp.int32)
    ka = _upper_kernel(B, bpw, NC, NS)
    kb = _leaf_kernel(B, bpw, NC)
    T0p = jnp.pad(T0, ((0, 0), (0, PAD - DIM)))
    part = ka(idx32, T1, T2, T3, P0, P1, P2)
    return kb(idx32, T0p, part)
